# Initial kernel scaffold; baseline (speedup 1.0000x reference)
import jax
import jax.numpy as jnp
from jax.experimental import pallas as pl


def kernel(users_feature, items_feature, edge_index, W_root0, W_rel0, b0, W_root1, W_rel1, b1):
    raise NotImplementedError("write your pallas kernel here")


# speedup vs baseline: 2.8237x; 2.8237x over previous
import functools

import jax
import jax.numpy as jnp
from jax import lax
from jax.experimental import pallas as pl
from jax.experimental.pallas import tpu as pltpu
from jax.experimental.pallas import tpu_sc as plsc

D = 64
CH = 128
BLK = 16
EPB = CH * BLK
NSUB = 16
NCORE = 2


def _segsum_sc(f, src2, map2, half, acc_rows):
    rows_total = src2.shape[0]
    rows_per_sub = rows_total // NSUB
    nblocks = rows_per_sub // BLK
    rows_per_tile = acc_rows // NSUB
    ZR = 112
    nzcopies = rows_per_tile // ZR
    NBUF = 2

    mesh = plsc.VectorSubcoreMesh(core_axis_name="c", subcore_axis_name="s")

    @functools.partial(
        pl.kernel,
        mesh=mesh,
        compiler_params=pltpu.CompilerParams(use_tc_tiling_on_sc=False),
        out_type=jax.ShapeDtypeStruct((NCORE, acc_rows, D), jnp.float32),
        scratch_types=[
            pltpu.VMEM((BLK, CH), jnp.int32),
            pltpu.VMEM((BLK, CH), jnp.int32),
            pltpu.VMEM((NBUF, CH, D), jnp.float32),
            pltpu.VMEM((ZR, D), jnp.float32),
            pltpu.VMEM_SHARED((acc_rows, D), jnp.float32),
            pltpu.SemaphoreType.DMA,
        ],
    )
    def seg_kernel(f_hbm, src_hbm, map_hbm, out_hbm,
                   src_v, map_v, rows_v, zero_v, acc_sh, gsem):
        c = lax.axis_index("c")
        s = lax.axis_index("s")

        def zfill(i, _):
            def zfill2(k, _):
                zero_v[i, pl.ds(k * 16, 16)] = jnp.zeros((16,), jnp.float32)
                return 0
            return lax.fori_loop(0, D // 16, zfill2, 0)
        lax.fori_loop(0, ZR, zfill, 0)

        def zcopy(t, _):
            pltpu.sync_copy(zero_v,
                            acc_sh.at[pl.ds(s * rows_per_tile + t * ZR, ZR)])
            return 0
        lax.fori_loop(0, nzcopies, zcopy, 0)
        plsc.subcore_barrier()

        def blk_body(blk, _):
            row0 = s * rows_per_sub + blk * BLK
            pltpu.sync_copy(src_hbm.at[pl.ds(row0, BLK)], src_v)
            pltpu.sync_copy(map_hbm.at[c].at[pl.ds(row0, BLK)], map_v)

            cps = [None] * BLK
            cps[0] = pltpu.async_copy(f_hbm.at[src_v.at[0]], rows_v.at[0],
                                      gsem)
            for j in range(BLK):
                cps[j].wait()
                if j + 1 < BLK:
                    cps[j + 1] = pltpu.async_copy(
                        f_hbm.at[src_v.at[j + 1]],
                        rows_v.at[(j + 1) % NBUF], gsem)
                pltpu.sync_copy(rows_v.at[j % NBUF], acc_sh.at[map_v.at[j]],
                                add=True)
            return 0
        lax.fori_loop(0, nblocks, blk_body, 0)

        plsc.subcore_barrier()
        pltpu.sync_copy(acc_sh.at[pl.ds(s * rows_per_tile, rows_per_tile)],
                        out_hbm.at[c].at[pl.ds(s * rows_per_tile,
                                               rows_per_tile)])

    return seg_kernel(f, src2, map2)


def _dense_layer(f, agg, Wr, Wn, b8, acc):
    NR = f.shape[0]
    BR = 2000

    def body(f_ref, a_ref, wr_ref, wn_ref, b_ref, acc_ref, fout_ref, aout_ref):
        x = f_ref[...]
        y = jnp.dot(x, wr_ref[...], preferred_element_type=jnp.float32,
                    precision=lax.Precision.HIGHEST)
        y = y + jnp.dot(a_ref[...], wn_ref[...],
                        preferred_element_type=jnp.float32,
                        precision=lax.Precision.HIGHEST)
        y = y + b_ref[0:1, :]
        fout_ref[...] = y
        nrm = jnp.sqrt(jnp.sum(y * y, axis=1, keepdims=True))
        aout_ref[...] = acc_ref[...] + y / jnp.maximum(nrm, 1e-12)

    return pl.pallas_call(
        body,
        grid=(NR // BR,),
        in_specs=[
            pl.BlockSpec((BR, D), lambda i: (i, 0)),
            pl.BlockSpec((BR, D), lambda i: (i, 0)),
            pl.BlockSpec((D, D), lambda i: (0, 0)),
            pl.BlockSpec((D, D), lambda i: (0, 0)),
            pl.BlockSpec((8, D), lambda i: (0, 0)),
            pl.BlockSpec((BR, D), lambda i: (i, 0)),
        ],
        out_specs=[
            pl.BlockSpec((BR, D), lambda i: (i, 0)),
            pl.BlockSpec((BR, D), lambda i: (i, 0)),
        ],
        out_shape=[
            jax.ShapeDtypeStruct((NR, D), jnp.float32),
            jax.ShapeDtypeStruct((NR, D), jnp.float32),
        ],
    )(f, agg, Wr, Wn, b8, acc)


def kernel(users_feature, items_feature, edge_index,
           W_root0, W_rel0, b0, W_root1, W_rel1, b1):
    feats = jnp.concatenate([users_feature, items_feature], axis=0)
    n_nodes = feats.shape[0]
    half = n_nodes // 2
    rows_per_tile = -(-(half + 8) // (16 * 112)) * 112
    acc_rows = rows_per_tile * 16

    src = edge_index[0]
    dst = edge_index[1]
    E = src.shape[0]
    per_sub = -(-E // (NSUB * EPB)) * EPB
    E_pad = per_sub * NSUB
    pad = E_pad - E
    src_p = jnp.concatenate([src, jnp.zeros((pad,), jnp.int32)])
    dst_p = jnp.concatenate([dst, jnp.full((pad,), n_nodes, jnp.int32)])
    src2 = src_p.reshape(E_pad // CH, CH)
    lo = jnp.arange(NCORE, dtype=jnp.int32)[:, None] * half
    t = dst_p[None, :] - lo
    map2 = jnp.where((t >= 0) & (t < half), t, half).reshape(
        NCORE, E_pad // CH, CH)

    b0b = jnp.broadcast_to(b0.reshape(1, D), (8, D))
    b1b = jnp.broadcast_to(b1.reshape(1, D), (8, D))

    f = feats
    acc = feats
    for (Wr, Wn, bb) in ((W_root0, W_rel0, b0b), (W_root1, W_rel1, b1b)):
        agg2 = _segsum_sc(f, src2, map2, half, acc_rows)
        agg = jnp.concatenate([agg2[0, :half], agg2[1, :half]], axis=0)
        f, acc = _dense_layer(f, agg, Wr, Wn, bb, acc)
    return acc


# speedup vs baseline: 3.2176x; 1.1395x over previous
import functools

import jax
import jax.numpy as jnp
from jax import lax
from jax.experimental import pallas as pl
from jax.experimental.pallas import tpu as pltpu
from jax.experimental.pallas import tpu_sc as plsc

D = 64
CH = 128
BLK = 16
EPB = CH * BLK
NSUB = 16
NCORE = 2


def _segsum_sc(f, src2, map2, half, acc_rows):
    rows_total = src2.shape[0]
    rows_per_sub = rows_total // NSUB
    nblocks = rows_per_sub // BLK
    rows_per_tile = acc_rows // NSUB
    ZR = 56
    nzcopies = rows_per_tile // ZR
    NBUF = 2
    G = 2

    mesh = plsc.VectorSubcoreMesh(core_axis_name="c", subcore_axis_name="s")

    @functools.partial(
        pl.kernel,
        mesh=mesh,
        compiler_params=pltpu.CompilerParams(use_tc_tiling_on_sc=False),
        out_type=jax.ShapeDtypeStruct((NCORE, acc_rows, D), jnp.float32),
        scratch_types=[
            pltpu.VMEM((2, BLK, CH), jnp.int32),
            pltpu.VMEM((2, BLK, CH), jnp.int32),
            pltpu.VMEM((NBUF, CH, D), jnp.float32),
            pltpu.VMEM((ZR, D), jnp.float32),
            pltpu.VMEM_SHARED((acc_rows, D), jnp.float32),
            pltpu.SemaphoreType.DMA,
            pltpu.SemaphoreType.DMA,
            pltpu.SemaphoreType.DMA((2,)),
        ],
    )
    def seg_kernel(f_hbm, src_hbm, map_hbm, out_hbm,
                   src_v, map_v, rows_v, zero_v, acc_sh,
                   gsem, ssem, isem):
        c = lax.axis_index("c")
        s = lax.axis_index("s")

        def zfill(i, _):
            def zfill2(k, _):
                zero_v[i, pl.ds(k * 16, 16)] = jnp.zeros((16,), jnp.float32)
                return 0
            return lax.fori_loop(0, D // 16, zfill2, 0)
        lax.fori_loop(0, ZR, zfill, 0)

        def zcopy(t, _):
            pltpu.sync_copy(zero_v,
                            acc_sh.at[pl.ds(s * rows_per_tile + t * ZR, ZR)])
            return 0
        lax.fori_loop(0, nzcopies, zcopy, 0)
        plsc.subcore_barrier()

        base = s * rows_per_sub

        def prefetch(blk, par):
            pltpu.async_copy(src_hbm.at[pl.ds(base + blk * BLK, BLK)],
                             src_v.at[par], isem.at[par])
            pltpu.async_copy(map_hbm.at[c].at[pl.ds(base + blk * BLK, BLK)],
                             map_v.at[par], isem.at[par])

        prefetch(0, 0)

        def blk_body(blk, _):
            par = lax.rem(blk, 2)
            for buf in (src_v, map_v):
                pltpu.make_async_copy(src_hbm.at[pl.ds(0, BLK)], buf.at[par],
                                      isem.at[par]).wait()
            sv = src_v.at[par]
            mv = map_v.at[par]

            @pl.when(blk + 1 < nblocks)
            def _():
                prefetch(blk + 1, 1 - par)

            gcp = [None] * BLK
            scp = [None] * BLK
            for j in range(G):
                gcp[j] = pltpu.async_copy(f_hbm.at[sv.at[j]],
                                          rows_v.at[j % NBUF], gsem)
            for j in range(BLK):
                gcp[j].wait()
                scp[j] = pltpu.async_copy(rows_v.at[j % NBUF],
                                          acc_sh.at[mv.at[j]], ssem,
                                          add=True)
                nj = j + G
                if nj < BLK:
                    if nj - NBUF >= 0:
                        scp[nj - NBUF].wait()
                    gcp[nj] = pltpu.async_copy(f_hbm.at[sv.at[nj]],
                                               rows_v.at[nj % NBUF], gsem)
            for j in range(max(BLK - NBUF, 0), BLK):
                scp[j].wait()
            return 0
        lax.fori_loop(0, nblocks, blk_body, 0)

        plsc.subcore_barrier()
        pltpu.sync_copy(acc_sh.at[pl.ds(s * rows_per_tile, rows_per_tile)],
                        out_hbm.at[c].at[pl.ds(s * rows_per_tile,
                                               rows_per_tile)])

    return seg_kernel(f, src2, map2)


def _dense_layer(f, agg, Wr, Wn, b8, acc):
    NR = f.shape[0]
    BR = 2000

    def body(f_ref, a_ref, wr_ref, wn_ref, b_ref, acc_ref, fout_ref, aout_ref):
        x = f_ref[...]
        y = jnp.dot(x, wr_ref[...], preferred_element_type=jnp.float32,
                    precision=lax.Precision.HIGHEST)
        y = y + jnp.dot(a_ref[...], wn_ref[...],
                        preferred_element_type=jnp.float32,
                        precision=lax.Precision.HIGHEST)
        y = y + b_ref[0:1, :]
        fout_ref[...] = y
        nrm = jnp.sqrt(jnp.sum(y * y, axis=1, keepdims=True))
        aout_ref[...] = acc_ref[...] + y / jnp.maximum(nrm, 1e-12)

    return pl.pallas_call(
        body,
        grid=(NR // BR,),
        in_specs=[
            pl.BlockSpec((BR, D), lambda i: (i, 0)),
            pl.BlockSpec((BR, D), lambda i: (i, 0)),
            pl.BlockSpec((D, D), lambda i: (0, 0)),
            pl.BlockSpec((D, D), lambda i: (0, 0)),
            pl.BlockSpec((8, D), lambda i: (0, 0)),
            pl.BlockSpec((BR, D), lambda i: (i, 0)),
        ],
        out_specs=[
            pl.BlockSpec((BR, D), lambda i: (i, 0)),
            pl.BlockSpec((BR, D), lambda i: (i, 0)),
        ],
        out_shape=[
            jax.ShapeDtypeStruct((NR, D), jnp.float32),
            jax.ShapeDtypeStruct((NR, D), jnp.float32),
        ],
    )(f, agg, Wr, Wn, b8, acc)


def kernel(users_feature, items_feature, edge_index,
           W_root0, W_rel0, b0, W_root1, W_rel1, b1):
    feats = jnp.concatenate([users_feature, items_feature], axis=0)
    n_nodes = feats.shape[0]
    half = n_nodes // 2
    rows_per_tile = -(-(half + 64) // (16 * 56)) * 56
    acc_rows = rows_per_tile * 16

    src = edge_index[0]
    dst = edge_index[1]
    E = src.shape[0]
    per_sub = -(-E // (NSUB * EPB)) * EPB
    E_pad = per_sub * NSUB
    pad = E_pad - E
    src_p = jnp.concatenate([src, jnp.zeros((pad,), jnp.int32)])
    dst_p = jnp.concatenate([dst, jnp.full((pad,), n_nodes, jnp.int32)])
    src2 = src_p.reshape(E_pad // CH, CH)
    lo = jnp.arange(NCORE, dtype=jnp.int32)[:, None] * half
    t = dst_p[None, :] - lo
    dummy = half + (jnp.arange(E_pad, dtype=jnp.int32)[None, :] & 63)
    map2 = jnp.where((t >= 0) & (t < half), t, dummy).reshape(
        NCORE, E_pad // CH, CH)

    b0b = jnp.broadcast_to(b0.reshape(1, D), (8, D))
    b1b = jnp.broadcast_to(b1.reshape(1, D), (8, D))

    f = feats
    acc = feats
    for (Wr, Wn, bb) in ((W_root0, W_rel0, b0b), (W_root1, W_rel1, b1b)):
        agg2 = _segsum_sc(f, src2, map2, half, acc_rows)
        agg = jnp.concatenate([agg2[0, :half], agg2[1, :half]], axis=0)
        f, acc = _dense_layer(f, agg, Wr, Wn, bb, acc)
    return acc
